# baseline (speedup 1.0000x reference)
import jax
import jax.numpy as jnp
from jax.experimental import pallas as pl

N_NODES = 10000
N_EDGES = 160000
DIM_IN = 384
DIM_H = 384
DIM_OUT = 384
NUM_HEADS = 8


def _mm_body(x_ref, w_ref, o_ref):
    o_ref[...] = jnp.dot(x_ref[...], w_ref[...],
                         preferred_element_type=jnp.float32)


def _mm(x, w, bm=1000):
    m, k = x.shape
    _, n = w.shape
    return pl.pallas_call(
        _mm_body,
        grid=(m // bm,),
        in_specs=[
            pl.BlockSpec((bm, k), lambda i: (i, 0)),
            pl.BlockSpec((k, n), lambda i: (0, 0)),
        ],
        out_specs=pl.BlockSpec((bm, n), lambda i: (i, 0)),
        out_shape=jax.ShapeDtypeStruct((m, n), jnp.float32),
    )(x, w)


def _gat_layer(x, src, dst, Wl, Wr, a, num_nodes):
    H, D = a.shape
    hs = _mm(x, Wl).reshape(-1, H, D)
    hd = _mm(x, Wr).reshape(-1, H, D)
    e = jax.nn.leaky_relu(hs[src] + hd[dst], negative_slope=0.2)
    logits = jnp.einsum('ehd,hd->eh', e, a)
    m = jax.ops.segment_max(logits, dst, num_segments=num_nodes)
    m = jnp.where(jnp.isfinite(m), m, 0.0)
    ex = jnp.exp(logits - m[dst])
    denom = jax.ops.segment_sum(ex, dst, num_segments=num_nodes)
    alpha = ex / (denom[dst] + 1e-9)
    out = jax.ops.segment_sum(alpha[:, :, None] * hs[src], dst,
                              num_segments=num_nodes)
    return out


def kernel(x, edge_index, Wl1, Wr1, a1, Wl2, Wr2, a2):
    N = x.shape[0]
    src = edge_index[0]
    dst = edge_index[1]
    g = _gat_layer(x, src, dst, Wl1, Wr1, a1, N)
    g = g.reshape(N, NUM_HEADS * DIM_H)
    g = jax.nn.elu(g)
    g = _gat_layer(g, src, dst, Wl2, Wr2, a2, N)
    return g.mean(axis=1)
